# Initial kernel scaffold; baseline (speedup 1.0000x reference)
import jax
import jax.numpy as jnp
from jax.experimental import pallas as pl


def kernel(noisy, deno, fflow, bflow):
    raise NotImplementedError("write your pallas kernel here")


# speedup vs baseline: 6.0948x; 6.0948x over previous
import functools

import jax
import jax.numpy as jnp
import numpy as np
from jax import lax
from jax.experimental import pallas as pl
from jax.experimental.pallas import tpu as pltpu
from jax.experimental.pallas import tpu_sc as plsc

WS = 9
WT = 1
PS = 7
K = 10
STRIDE0 = 4
T, C, H, W = 5, 3, 128, 128
HP = H + PS - 1
PLANE = HP * HP
FRAME = C * PLANE
FRAME_PAD = 53888
NH = H // STRIDE0
NWIN = WS * WS
NWIN_PAD = 96
NDT = 2 * WT + 1
NCHUNK = NWIN_PAD // 16
NSLOT = NDT * NWIN_PAD
NVREG = NSLOT // 16
NTILES = 32

_INF = float(np.inf)

NELEM = C * PS * PS


def _candidate_bases(fflow, bflow):
    qh = jnp.arange(0, H, STRIDE0)
    qw = jnp.arange(0, W, STRIDE0)
    off = jnp.arange(WS) - WS // 2
    owi, owj = jnp.meshgrid(off, off, indexing="ij")
    owi = owi.reshape(-1)
    owj = owj.reshape(-1)
    blocks = []
    for t in range(T):
        for dt in range(-WT, WT + 1):
            if dt == 0:
                fh = jnp.zeros((NH, NH), jnp.float32)
                fw = jnp.zeros((NH, NH), jnp.float32)
            elif dt > 0:
                fl = fflow[t]
                fw = fl[0][qh[:, None], qw[None, :]] * dt
                fh = fl[1][qh[:, None], qw[None, :]] * dt
            else:
                fl = bflow[t]
                fw = fl[0][qh[:, None], qw[None, :]] * (-dt)
                fh = fl[1][qh[:, None], qw[None, :]] * (-dt)
            ch = jnp.clip(
                jnp.round(qh[:, None].astype(jnp.float32) + fh).astype(jnp.int32)[:, :, None]
                + owi[None, None, :], 0, H - 1)
            cw = jnp.clip(
                jnp.round(qw[None, :].astype(jnp.float32) + fw).astype(jnp.int32)[:, :, None]
                + owj[None, None, :], 0, W - 1)
            blocks.append(ch * HP + cw)
    cb = jnp.stack(blocks).reshape(T * NDT, NH, NH, NWIN)
    cb = jnp.pad(cb, ((0, 0), (0, 0), (0, 0), (0, NWIN_PAD - NWIN)))
    return cb.reshape(T * NDT * NH, NH, NWIN_PAD).astype(jnp.int32)


def _sc_body(vp_hbm, cb_hbm, out_hbm,
             frame_a, frame_b, cb_v, dist, stage):
    tid = lax.axis_index("s") * 2 + lax.axis_index("c")
    lane = lax.broadcasted_iota(jnp.int32, (16,), 0)

    def t_body(t, total):
        pltpu.sync_copy(vp_hbm.at[t], frame_a)
        for dtidx in range(NDT):
            if dtidx == 0:
                tf = jnp.maximum(t - 1, 0)
                pltpu.sync_copy(vp_hbm.at[tf], frame_b)
                gref = frame_b
            elif dtidx == 1:
                gref = frame_a
            else:
                tf = jnp.minimum(t + 1, T - 1)
                pltpu.sync_copy(vp_hbm.at[tf], frame_b)
                gref = frame_b
            pltpu.sync_copy(cb_hbm.at[(t * NDT + dtidx) * NH + tid], cb_v)

            def q_body(q, _, gref=gref, dtidx=dtidx):
                qbase = tid * (STRIDE0 * HP) + q * STRIDE0
                bases = [cb_v[q, pl.ds(k * 16, 16)] for k in range(NCHUNK)]

                def e_body(e, accs):
                    ci = e // (PS * PS)
                    r = e % (PS * PS)
                    off = ci * PLANE + (r // PS) * HP + (r % PS)
                    qv = frame_a[pl.ds(qbase + off, 16)][0]
                    out = []
                    for k in range(NCHUNK):
                        cv = plsc.load_gather(gref, [bases[k] + off])
                        dd = cv - qv
                        out.append(accs[k] + dd * dd)
                    return tuple(out)

                accs = lax.fori_loop(
                    0, NELEM, e_body,
                    tuple(jnp.zeros((16,), jnp.float32) for _ in range(NCHUNK)))
                for k in range(NCHUNK):
                    vec = accs[k]
                    if (k + 1) * 16 > NWIN:
                        vec = jnp.where(lane < NWIN - k * 16, vec, _INF)
                    dist[q, pl.ds(dtidx * NWIN_PAD + k * 16, 16)] = vec
                return 0

            lax.fori_loop(0, NH, q_body, 0)

        def topk_body(q, tot):
            vs = [jnp.sort(dist[q, pl.ds(i * 16, 16)]) for i in range(NVREG)]
            while len(vs) > 1:
                nxt = []
                for i in range(0, len(vs) - 1, 2):
                    nxt.append(jnp.sort(jnp.minimum(vs[i], lax.rev(vs[i + 1], (0,)))))
                if len(vs) % 2:
                    nxt.append(vs[-1])
                vs = nxt
            qsum = jnp.sum(jnp.where(lane < K, vs[0], jnp.float32(0.0)))
            return tot + qsum

        return lax.fori_loop(0, NH, topk_body, total)

    total = lax.fori_loop(0, T, t_body, jnp.float32(0.0))
    stage[...] = jnp.where(lane < 1, total, jnp.float32(0.0))
    pltpu.sync_copy(stage, out_hbm.at[tid])


@jax.jit
def kernel(noisy, deno, fflow, bflow):
    del deno
    vid = noisy[0]
    vp = jnp.pad(vid, ((0, 0), (0, 0), (PS // 2, PS // 2), (PS // 2, PS // 2)),
                 mode="edge")
    vp_flat = jnp.pad(vp.reshape(T, FRAME), ((0, 0), (0, FRAME_PAD - FRAME)))
    cb = _candidate_bases(fflow[0], bflow[0])

    mesh = plsc.VectorSubcoreMesh(core_axis_name="c", subcore_axis_name="s")
    run = functools.partial(
        pl.kernel,
        mesh=mesh,
        compiler_params=pltpu.CompilerParams(needs_layout_passes=False),
        out_type=jax.ShapeDtypeStruct((NTILES, 16), jnp.float32),
        scratch_types=[
            pltpu.VMEM((FRAME_PAD,), jnp.float32),
            pltpu.VMEM((FRAME_PAD,), jnp.float32),
            pltpu.VMEM((NH, NWIN_PAD), jnp.int32),
            pltpu.VMEM((NH, NSLOT), jnp.float32),
            pltpu.VMEM((16,), jnp.float32),
        ],
    )(_sc_body)
    partials = run(vp_flat, cb)
    return jnp.sum(partials) / jnp.float32(T * NH * NH * K)
